# Initial kernel scaffold; baseline (speedup 1.0000x reference)
import jax
import jax.numpy as jnp
from jax.experimental import pallas as pl


def kernel(pos, x, idx, Wl1, bl1, Wl2, bl2, Wg1, bg1, Wg2, bg2, Wh, bh):
    raise NotImplementedError("write your pallas kernel here")


# speedup vs baseline: 2.1229x; 2.1229x over previous
import functools

import jax
import jax.numpy as jnp
from jax.experimental import pallas as pl

_S = 10000
_S_PAD = 10752
_W = 128
_B = 3200
_SENT = 2 ** 30
_NEG = float("-inf")


def _run_windows(idx_col, body, init):
    def cond(c):
        return c[0] < _SENT

    def step(c):
        base_raw, carry = c
        base = (base_raw // 8) * 8
        cols = base + jax.lax.broadcasted_iota(jnp.int32, (1, _W), 1)
        mask = idx_col == cols
        carry = body(base, mask, carry)
        nxt = jnp.min(jnp.where(idx_col >= base + _W, idx_col, _SENT))
        return (nxt, carry)

    first = jnp.min(idx_col)
    return jax.lax.while_loop(cond, step, (first, init))[1]


def _leader_mask(idx_col):
    prev = jnp.concatenate(
        [jnp.full((1, 1), -1, idx_col.dtype), idx_col[:-1, :]], axis=0)
    return idx_col != prev


def _seg_suffix_max(vals, idx_col):
    b = vals.shape[0]
    x = vals
    ii = idx_col
    s = 1
    while s < b:
        xs = jnp.concatenate(
            [x[s:, :], jnp.full((s, x.shape[1]), _NEG, x.dtype)], axis=0)
        is_ = jnp.concatenate(
            [ii[s:, :], jnp.full((s, 1), -2, ii.dtype)], axis=0)
        x = jnp.where(ii == is_, jnp.maximum(x, xs), x)
        s *= 2
    return x


def _tmm(a, b):
    return jax.lax.dot_general(
        a, b, (((0,), (0,)), ((), ())), preferred_element_type=jnp.float32)


def _stats_kernel(idx_ref, pos_ref, stats_ref, *, nblocks):
    i = pl.program_id(0)

    @pl.when(i == 0)
    def _():
        stats_ref[:] = jnp.zeros_like(stats_ref)

    idx = idx_ref[:]
    pos = pos_ref[:]
    data = jnp.concatenate(
        [pos, jnp.ones((pos.shape[0], 1), jnp.float32)], axis=1)

    def body(base, mask, carry):
        ohf = mask.astype(jnp.float32)
        stats_ref[pl.ds(base, _W), :] += _tmm(ohf, data)
        return carry

    _run_windows(idx, body, 0)

    @pl.when(i == nblocks - 1)
    def _():
        s = stats_ref[:]
        cnt = s[:, 3:4]
        cen = s[:, 0:3] / jnp.maximum(cnt, 1.0)
        stats_ref[:] = jnp.concatenate([cen, cnt], axis=1)


def _radius_kernel(idx_ref, pos_ref, stats_ref, rad_ref):
    i = pl.program_id(0)

    @pl.when(i == 0)
    def _():
        rad_ref[:] = jnp.full_like(rad_ref, _NEG)

    idx = idx_ref[:]
    pos = pos_ref[:]
    b = pos.shape[0]

    def gather(base, mask, acc):
        ohf = mask.astype(jnp.float32)
        sw = stats_ref[pl.ds(base, _W), :]
        return acc + jnp.dot(ohf, sw, preferred_element_type=jnp.float32)

    g = _run_windows(idx, gather, jnp.zeros((b, 4), jnp.float32))
    centered = pos - g[:, 0:3]
    dist = jnp.sqrt(
        jnp.sum(centered * centered, axis=1, keepdims=True) + 1e-12)

    leader = _leader_mask(idx)
    dmax = _seg_suffix_max(dist, idx)
    packed = jnp.concatenate(
        [dmax, jnp.ones((b, 1), jnp.float32)], axis=1)

    def scatter(base, mask, carry):
        ohl = jnp.where(mask & leader, 1.0, 0.0)
        out = _tmm(ohl, packed)
        upd = jnp.where(out[:, 1:2] > 0.0, out[:, 0:1], _NEG)
        rad_ref[pl.ds(base, _W), :] = jnp.maximum(
            rad_ref[pl.ds(base, _W), :], upd)
        return carry

    _run_windows(idx, scatter, 0)


def _point_kernel(idx_ref, pos_ref, x_ref, stats_ref, rad_ref,
                  wl1_ref, bl1_ref, wl2_ref, bl2_ref, xg_ref):
    i = pl.program_id(0)

    @pl.when(i == 0)
    def _():
        xg_ref[:] = jnp.full_like(xg_ref, _NEG)

    idx = idx_ref[:]
    pos = pos_ref[:]
    b = pos.shape[0]

    def gather(base, mask, acc):
        ohf = mask.astype(jnp.float32)
        sw = stats_ref[pl.ds(base, _W), :]
        rw = jnp.maximum(rad_ref[pl.ds(base, _W), :], 1e-12)
        src = jnp.concatenate([sw[:, 0:3], rw], axis=1)
        return acc + jnp.dot(ohf, src, preferred_element_type=jnp.float32)

    g = _run_windows(idx, gather, jnp.zeros((b, 4), jnp.float32))
    centered = pos - g[:, 0:3]
    pos_n = centered / g[:, 3:4]

    feat = jnp.concatenate([x_ref[:], pos_n], axis=1)
    h = jax.lax.dot_general(
        feat, wl1_ref[:], (((1,), (1,)), ((), ())),
        preferred_element_type=jnp.float32) + bl1_ref[0:1, :]
    h = jnp.maximum(h, 0.0)
    h = jax.lax.dot_general(
        h, wl2_ref[:], (((1,), (1,)), ((), ())),
        preferred_element_type=jnp.float32) + bl2_ref[0:1, :]
    h = jnp.maximum(h, 0.0)

    leader = _leader_mask(idx)
    hmax = _seg_suffix_max(h, idx)
    packed = jnp.concatenate(
        [hmax, jnp.ones((b, 1), jnp.float32)], axis=1)

    def scatter(base, mask, carry):
        ohl = jnp.where(mask & leader, 1.0, 0.0)
        out = _tmm(ohl, packed)
        upd = jnp.where(out[:, 128:129] > 0.0, out[:, 0:128], _NEG)
        xg_ref[pl.ds(base, _W), :] = jnp.maximum(
            xg_ref[pl.ds(base, _W), :], upd)
        return carry

    _run_windows(idx, scatter, 0)


def _head_kernel(xg_ref, rad_ref, wg1_ref, bg1_ref, wg2_ref, bg2_ref,
                 wh_ref, bh_ref, out_ref):
    xg = xg_ref[:]
    xg = jnp.where(jnp.isfinite(xg), xg, 0.0)
    diam = 2.0 * jnp.maximum(rad_ref[:], 1e-12)
    z = jnp.concatenate([xg, diam], axis=1)
    z = jax.lax.dot_general(
        z, wg1_ref[:], (((1,), (1,)), ((), ())),
        preferred_element_type=jnp.float32) + bg1_ref[0:1, :]
    z = jnp.maximum(z, 0.0)
    z = jax.lax.dot_general(
        z, wg2_ref[:], (((1,), (1,)), ((), ())),
        preferred_element_type=jnp.float32) + bg2_ref[0:1, :]
    z = jnp.maximum(z, 0.0)
    out_ref[:] = jax.lax.dot_general(
        z, wh_ref[:], (((1,), (1,)), ((), ())),
        preferred_element_type=jnp.float32) + bh_ref[0:1, :]


def _full(shape):
    return pl.BlockSpec(shape, lambda i: (0, 0))


def kernel(pos, x, idx, Wl1, bl1, Wl2, bl2, Wg1, bg1, Wg2, bg2, Wh, bh):
    n = pos.shape[0]
    nb = n // _B
    idx2 = idx.astype(jnp.int32).reshape(n, 1)
    b2 = lambda v: jnp.broadcast_to(v.reshape(1, -1), (8, v.shape[0]))

    stats = pl.pallas_call(
        functools.partial(_stats_kernel, nblocks=nb),
        grid=(nb,),
        in_specs=[pl.BlockSpec((_B, 1), lambda i: (i, 0)),
                  pl.BlockSpec((_B, 3), lambda i: (i, 0))],
        out_specs=_full((_S_PAD, 4)),
        out_shape=jax.ShapeDtypeStruct((_S_PAD, 4), jnp.float32),
    )(idx2, pos)

    rad = pl.pallas_call(
        _radius_kernel,
        grid=(nb,),
        in_specs=[pl.BlockSpec((_B, 1), lambda i: (i, 0)),
                  pl.BlockSpec((_B, 3), lambda i: (i, 0)),
                  _full((_S_PAD, 4))],
        out_specs=_full((_S_PAD, 1)),
        out_shape=jax.ShapeDtypeStruct((_S_PAD, 1), jnp.float32),
    )(idx2, pos, stats)

    xg = pl.pallas_call(
        _point_kernel,
        grid=(nb,),
        in_specs=[pl.BlockSpec((_B, 1), lambda i: (i, 0)),
                  pl.BlockSpec((_B, 3), lambda i: (i, 0)),
                  pl.BlockSpec((_B, 16), lambda i: (i, 0)),
                  _full((_S_PAD, 4)),
                  _full((_S_PAD, 1)),
                  _full((64, 19)), _full((8, 64)),
                  _full((128, 64)), _full((8, 128))],
        out_specs=_full((_S_PAD, 128)),
        out_shape=jax.ShapeDtypeStruct((_S_PAD, 128), jnp.float32),
    )(idx2, pos, x, stats, rad, Wl1, b2(bl1), Wl2, b2(bl2))

    r = 1536
    out = pl.pallas_call(
        _head_kernel,
        grid=(_S_PAD // r,),
        in_specs=[pl.BlockSpec((r, 128), lambda i: (i, 0)),
                  pl.BlockSpec((r, 1), lambda i: (i, 0)),
                  _full((256, 129)), _full((8, 256)),
                  _full((128, 256)), _full((8, 128)),
                  _full((40, 128)), _full((8, 40))],
        out_specs=pl.BlockSpec((r, 40), lambda i: (i, 0)),
        out_shape=jax.ShapeDtypeStruct((_S_PAD, 40), jnp.float32),
    )(xg, rad, Wg1, b2(bg1), Wg2, b2(bg2), Wh, b2(bh))

    return out[:_S]
